# baseline (speedup 1.0000x reference)
import functools

import jax
import jax.numpy as jnp
from jax import lax
from jax.experimental import pallas as pl
from jax.experimental.pallas import tpu as pltpu
from jax.experimental.pallas import tpu_sc as plsc

N_NODE = 50000
EMB = 100
EMBP = 112
LAYERS = 3
BATCH = 1024
SEQ = 50
NNZ = 800000
BETA = 0.01

W = 64
NCHUNK = 2
NROWS = 50176
ROWS_PER_SUB = NROWS // 16
EDGES_PER_SUB = 50176
NNZ_PAD = 16 * EDGES_PER_SUB
EBLOCKS = NNZ_PAD // 128
BLOCKS_PER_SUB = EBLOCKS // 16
EBATCH = 2
BATCHES_PER_SUB = BLOCKS_PER_SUB // EBATCH

SEQ_TOT = BATCH * SEQ
SEQ_PAD = 53248
SEQ_BLOCKS = SEQ_PAD // 128
SEQ_BLOCKS_PER_W = SEQ_BLOCKS // 32

def _sc_mesh():
    return plsc.VectorSubcoreMesh(core_axis_name="c", subcore_axis_name="s",
                                  num_cores=2, num_subcores=16)


_SC_PARAMS = pltpu.CompilerParams(use_tc_tiling_on_sc=False,
                                  needs_layout_passes=False)


def _hyperconv_sc(e0, e1, row2d, col2d, val1d):
    f32 = jnp.float32
    bf16 = jnp.bfloat16
    i32 = jnp.int32
    out_t = jax.ShapeDtypeStruct((NCHUNK, LAYERS + 1, NROWS, W), bf16)

    NBUF = 3
    scratch_types = [pltpu.VMEM_SHARED((NROWS, W), bf16)]
    scratch_types += [pltpu.VMEM((EBATCH, 128), i32)] * NBUF
    scratch_types += [pltpu.VMEM((EBATCH, 128), i32)] * NBUF
    scratch_types += [pltpu.VMEM((EBATCH * 128,), f32)] * NBUF
    scratch_types += [pltpu.VMEM((EBATCH * 128, W), bf16)] * NBUF
    scratch_types += [pltpu.VMEM((64, W), bf16)]
    scratch_types += [pltpu.SemaphoreType.DMA] * (2 * NBUF)

    @functools.partial(
        pl.kernel,
        out_type=out_t,
        mesh=_sc_mesh(),
        scratch_types=scratch_types,
        compiler_params=_SC_PARAMS,
    )
    def k(e0_h, e1_h, row_h, col_h, val_h, o_h, *scr):
        acc = scr[0]
        colv = scr[1:1 + NBUF]
        rowv = scr[1 + NBUF:1 + 2 * NBUF]
        valv = scr[1 + 2 * NBUF:1 + 3 * NBUF]
        rows = scr[1 + 3 * NBUF:1 + 4 * NBUF]
        zbuf = scr[1 + 4 * NBUF]
        gsem = scr[2 + 4 * NBUF:2 + 4 * NBUF + NBUF]
        ssem = scr[2 + 4 * NBUF + NBUF:]
        core = lax.axis_index("c")
        sid = lax.axis_index("s")

        zv = jnp.zeros((32,), bf16)

        @pl.loop(0, 64)
        def _(i):
            zbuf[i, pl.ds(0, 32)] = zv
            zbuf[i, pl.ds(32, 32)] = zv

        def zero_acc():
            base = sid * ROWS_PER_SUB

            @pl.loop(0, ROWS_PER_SUB // 64)
            def _(i):
                pltpu.sync_copy(zbuf, acc.at[pl.ds(base + i * 64, 64)])

        def edge_pass(tab):
            def fire(i, b):
                blk = sid * BLOCKS_PER_SUB + i * EBATCH
                pltpu.sync_copy(col_h.at[pl.ds(blk, EBATCH)], colv[b])
                pltpu.sync_copy(row_h.at[pl.ds(blk, EBATCH)], rowv[b])
                pltpu.sync_copy(val_h.at[pl.ds(blk * 128, EBATCH * 128)],
                                valv[b])
                for j in range(EBATCH):
                    pltpu.async_copy(tab.at[colv[b].at[j]],
                                     rows[b].at[pl.ds(j * 128, 128)], gsem[b])

            def wait_scatter(b):
                for j in range(EBATCH):
                    pltpu.make_async_copy(
                        rows[b].at[pl.ds(j * 128, 128)],
                        acc.at[rowv[b].at[j]], ssem[b]).wait()

            def handle(b):
                for j in range(EBATCH):
                    pltpu.make_async_copy(
                        tab.at[colv[b].at[j]],
                        rows[b].at[pl.ds(j * 128, 128)], gsem[b]).wait()

                @pl.loop(0, EBATCH * 8)
                def _(g):
                    vv = valv[b][pl.ds(g * 16, 16)]
                    e0 = g * 16
                    for t in range(16):
                        s = jnp.full((16,), vv[t], f32)
                        sb = plsc.pack(s, s,
                                       format=plsc.PackFormat.INTERLEAVED)
                        rows[b][e0 + t, pl.ds(0, 32)] = (
                            rows[b][e0 + t, pl.ds(0, 32)] * sb)
                        rows[b][e0 + t, pl.ds(32, 32)] = (
                            rows[b][e0 + t, pl.ds(32, 32)] * sb)

                for j in range(EBATCH):
                    pltpu.async_copy(rows[b].at[pl.ds(j * 128, 128)],
                                     acc.at[rowv[b].at[j]], ssem[b], add=True)

            NB = BATCHES_PER_SUB
            fire(0, 0)
            fire(1, 1)

            @pl.loop(0, (NB + 3) // 3)
            def _(kk):
                for r in range(3):
                    i = kk * 3 + r

                    @pl.when(i < NB)
                    def _():
                        handle(r)

                    @pl.when(jnp.logical_and(i >= 1, i <= NB))
                    def _():
                        wait_scatter((r + 2) % 3)

                    @pl.when(i + 2 < NB)
                    def _():
                        fire(i + 2, (r + 2) % 3)

        def drain(out_h):
            base = sid * ROWS_PER_SUB
            pltpu.sync_copy(acc.at[pl.ds(base, ROWS_PER_SUB)],
                            out_h.at[pl.ds(base, ROWS_PER_SUB)])

        base = sid * ROWS_PER_SUB
        sl = pl.ds(base, ROWS_PER_SUB)

        @pl.when(core == 0)
        def _():
            pltpu.sync_copy(e0_h.at[sl], o_h.at[0, 0].at[sl])

        @pl.when(core == 1)
        def _():
            pltpu.sync_copy(e1_h.at[sl], o_h.at[1, 0].at[sl])

        plsc.subcore_barrier()

        @pl.loop(0, LAYERS)
        def _(layer):
            c = core
            zero_acc()
            plsc.subcore_barrier()
            edge_pass(o_h.at[c, layer])
            plsc.subcore_barrier()
            drain(o_h.at[c, layer + 1])
            plsc.subcore_barrier()

    return k(e0, e1, row2d, col2d, val1d)


def _gather_sc(tab, idx2d):
    f32 = jnp.float32

    @functools.partial(
        pl.kernel,
        out_type=jax.ShapeDtypeStruct((SEQ_PAD, EMBP), f32),
        mesh=_sc_mesh(),
        scratch_types=[
            pltpu.VMEM((128,), jnp.int32),
            pltpu.VMEM((128, EMBP), f32),
            pltpu.SemaphoreType.DMA,
        ],
        compiler_params=_SC_PARAMS,
    )
    def k(tab_h, idx_h, out_h, idxv, rowbuf, sem):
        core = lax.axis_index("c")
        sid = lax.axis_index("s")
        wid = sid * 2 + core

        @pl.loop(0, SEQ_BLOCKS_PER_W)
        def _(i):
            blk = wid * SEQ_BLOCKS_PER_W + i
            pltpu.sync_copy(idx_h.at[blk], idxv)
            pltpu.async_copy(tab_h.at[idxv], rowbuf, sem).wait()
            pltpu.sync_copy(rowbuf, out_h.at[pl.ds(blk * 128, 128)])

    return k(tab, idx2d)


def _assemble_tc(embP, O):
    RB = 2000

    def body(emb_ref, o_ref, out_ref):
        f32 = jnp.float32
        o = o_ref[...].astype(f32)
        sums = [o[c, 1] + o[c, 2] + o[c, 3] for c in range(NCHUNK)]
        cat = jnp.concatenate([sums[0], sums[1][:, :EMBP - W]], axis=1)
        out_ref[...] = emb_ref[...] + cat

    return pl.pallas_call(
        body,
        grid=(N_NODE // RB,),
        in_specs=[
            pl.BlockSpec((RB, EMBP), lambda i: (i, 0)),
            pl.BlockSpec((NCHUNK, LAYERS + 1, RB, W),
                         lambda i: (0, 0, i, 0)),
        ],
        out_specs=pl.BlockSpec((RB, EMBP), lambda i: (i, 0)),
        out_shape=jax.ShapeDtypeStruct((N_NODE, EMBP), jnp.float32),
    )(embP, O)


def _sess_sum_tc(seqg, si_col, session_len):
    BB = 128

    def body(sq_ref, nz_ref, len_ref, out_ref):
        f32 = jnp.float32
        sq = sq_ref[...] * (nz_ref[...] != 0).astype(f32)
        out_ref[...] = jnp.sum(sq.reshape(BB, SEQ, EMBP), axis=1) / len_ref[...]

    return pl.pallas_call(
        body,
        grid=(BATCH // BB,),
        in_specs=[
            pl.BlockSpec((BB * SEQ, EMBP), lambda i: (i, 0)),
            pl.BlockSpec((BB * SEQ, 1), lambda i: (i, 0)),
            pl.BlockSpec((BB, 1), lambda i: (i, 0)),
        ],
        out_specs=pl.BlockSpec((BB, EMBP), lambda i: (i, 0)),
        out_shape=jax.ShapeDtypeStruct((BATCH, EMBP), jnp.float32),
    )(seqg, si_col, session_len)


def _lineconv_tc(D, A, s):
    def body(d_ref, a_ref, s_ref, out_ref):
        f32 = jnp.float32
        da = jnp.dot(d_ref[...], a_ref[...], preferred_element_type=f32)
        c = s_ref[...]
        acc = c
        for _ in range(LAYERS):
            c = jnp.dot(da, c, preferred_element_type=f32)
            acc = acc + c
        out_ref[...] = acc

    return pl.pallas_call(
        body,
        out_shape=jax.ShapeDtypeStruct((BATCH, EMBP), jnp.float32),
    )(D, A, s)


def _attention_tc(seqg, rsi_col, mask_col, session_len, posP,
                  w1t, w1b, glu1P, b1P, glu2P, w2P):
    BB = 128

    def body(sq_ref, nz_ref, mk_ref, len_ref, pos_ref, w1t_ref, w1b_ref,
             g1_ref, b1_ref, g2_ref, w2_ref, out_ref):
        f32 = jnp.float32
        sq = sq_ref[...] * (nz_ref[...] != 0).astype(f32)
        sq3 = sq.reshape(BB, SEQ, EMBP)
        hs = jnp.sum(sq3, axis=1) / len_ref[...]
        pos_t = jnp.dot(pos_ref[...], w1t_ref[...], preferred_element_type=f32)
        t1 = jnp.dot(sq, w1b_ref[...], preferred_element_type=f32)
        nh = jnp.tanh(t1.reshape(BB, SEQ, EMBP) + pos_t[None])
        hsg = jnp.dot(hs, g2_ref[...], preferred_element_type=f32)
        g1 = jnp.dot(nh.reshape(BB * SEQ, EMBP), g1_ref[...],
                     preferred_element_type=f32)
        g = jax.nn.sigmoid(g1.reshape(BB, SEQ, EMBP) + b1_ref[...][None]
                           + hsg[:, None, :])
        beta = jnp.sum(g * w2_ref[...][None], axis=-1, keepdims=True)
        beta = beta * mk_ref[...].reshape(BB, SEQ, 1)
        out_ref[...] = jnp.sum(beta * sq3, axis=1)

    return pl.pallas_call(
        body,
        grid=(BATCH // BB,),
        in_specs=[
            pl.BlockSpec((BB * SEQ, EMBP), lambda i: (i, 0)),
            pl.BlockSpec((BB * SEQ, 1), lambda i: (i, 0)),
            pl.BlockSpec((BB * SEQ, 1), lambda i: (i, 0)),
            pl.BlockSpec((BB, 1), lambda i: (i, 0)),
            pl.BlockSpec((SEQ, EMBP), lambda i: (0, 0)),
            pl.BlockSpec((EMBP, EMBP), lambda i: (0, 0)),
            pl.BlockSpec((EMBP, EMBP), lambda i: (0, 0)),
            pl.BlockSpec((EMBP, EMBP), lambda i: (0, 0)),
            pl.BlockSpec((1, EMBP), lambda i: (0, 0)),
            pl.BlockSpec((EMBP, EMBP), lambda i: (0, 0)),
            pl.BlockSpec((1, EMBP), lambda i: (0, 0)),
        ],
        out_specs=pl.BlockSpec((BB, EMBP), lambda i: (i, 0)),
        out_shape=jax.ShapeDtypeStruct((BATCH, EMBP), jnp.float32),
    )(seqg, rsi_col, mask_col, session_len, posP, w1t, w1b, glu1P, b1P,
      glu2P, w2P)


def _loss_tc(se, sl, Pr, Pc):
    def body(se_ref, sl_ref, pr_ref, pc_ref, out_ref):
        f32 = jnp.float32
        se_v = se_ref[...]
        sl_v = sl_ref[...]
        corrupt = jnp.dot(
            jnp.dot(pr_ref[...], se_v, preferred_element_type=f32),
            pc_ref[...], preferred_element_type=f32)
        pos = jnp.sum(se_v * sl_v, axis=1, keepdims=True)
        neg = jnp.sum(sl_v * corrupt, axis=1, keepdims=True)
        term = (-jnp.log(1e-08 + jax.nn.sigmoid(pos))
                - jnp.log(1e-08 + (1.0 - jax.nn.sigmoid(neg))))
        out_ref[...] = (BETA * jnp.sum(term)).reshape(1, 1)

    return pl.pallas_call(
        body,
        out_shape=jax.ShapeDtypeStruct((1, 1), jnp.float32),
    )(se, sl, Pr, Pc)


def kernel(session_item, session_len, D, A, reversed_sess_item, mask,
           adj_row, adj_col, adj_val, embedding, pos_embedding,
           w_1, w_2, glu1_w, glu1_b, glu2_w):
    f32 = jnp.float32
    i32 = jnp.int32

    emb128 = jnp.pad(embedding, ((0, 0), (0, 4 * W - EMB)))
    emb128b = jnp.pad(emb128.astype(jnp.bfloat16),
                      ((0, NROWS - N_NODE), (0, 0)))
    e_chunks = [emb128b[:, c * W:(c + 1) * W] for c in range(NCHUNK)]
    embP = emb128[:, :EMBP]

    padE = NNZ_PAD - NNZ
    rowp = jnp.concatenate(
        [adj_row, jnp.full((padE,), N_NODE, i32)]).reshape(EBLOCKS, 128)
    colp = jnp.concatenate(
        [adj_col, jnp.zeros((padE,), i32)]).reshape(EBLOCKS, 128)
    valp = jnp.concatenate([adj_val, jnp.zeros((padE,), f32)])

    idx_pad = jnp.zeros((SEQ_PAD - SEQ_TOT,), i32)
    si_flat = session_item.reshape(-1)
    si_idx = jnp.concatenate(
        [jnp.maximum(si_flat - 1, 0), idx_pad]).reshape(SEQ_BLOCKS, 128)
    rsi_flat = reversed_sess_item.reshape(-1)
    rsi_idx = jnp.concatenate(
        [jnp.maximum(rsi_flat - 1, 0), idx_pad]).reshape(SEQ_BLOCKS, 128)

    pad12 = EMBP - EMB
    w1t = jnp.pad(w_1[:EMB], ((0, pad12), (0, pad12)))
    w1b = jnp.pad(w_1[EMB:], ((0, pad12), (0, pad12)))
    glu1P = jnp.pad(glu1_w, ((0, pad12), (0, pad12)))
    glu2P = jnp.pad(glu2_w, ((0, pad12), (0, pad12)))
    b1P = jnp.pad(glu1_b, (0, pad12)).reshape(1, EMBP)
    w2P = jnp.pad(w_2[:, 0], (0, pad12)).reshape(1, EMBP)
    posP = jnp.pad(pos_embedding[:SEQ], ((0, 0), (0, pad12)))

    pr = jax.random.permutation(jax.random.key(123), BATCH)
    pc = jax.random.permutation(jax.random.key(456), EMB)
    Pr = jax.nn.one_hot(pr, BATCH, dtype=f32)
    Pc = jnp.pad(jax.nn.one_hot(pc, EMBP, dtype=f32).T, ((0, 0), (0, pad12)))

    seq2g = _gather_sc(embP, si_idx)
    hc_out = _hyperconv_sc(e_chunks[0], e_chunks[1], rowp, colp, valp)

    s = _sess_sum_tc(seq2g, si_flat.reshape(-1, 1), session_len)
    sess_lg = _lineconv_tc(D, A, s)

    item_hgP = _assemble_tc(embP, hc_out)
    seqhg = _gather_sc(item_hgP, rsi_idx)
    sess_emb = _attention_tc(seqhg, rsi_flat.reshape(-1, 1),
                             mask.reshape(-1, 1), session_len, posP,
                             w1t, w1b, glu1P, b1P, glu2P, w2P)
    loss = _loss_tc(sess_emb, sess_lg, Pr, Pc)

    return item_hgP[:, :EMB], sess_emb[:, :EMB], loss.reshape(())


# speedup vs baseline: 1.0362x; 1.0362x over previous
import functools

import jax
import jax.numpy as jnp
from jax import lax
from jax.experimental import pallas as pl
from jax.experimental.pallas import tpu as pltpu
from jax.experimental.pallas import tpu_sc as plsc

N_NODE = 50000
EMB = 100
EMBP = 112
LAYERS = 3
BATCH = 1024
SEQ = 50
NNZ = 800000
BETA = 0.01

W = 32
NCHUNK = 4
NROWS = 50176
ROWS_PER_SUB = NROWS // 16
EDGES_PER_SUB = 50176
NNZ_PAD = 16 * EDGES_PER_SUB
EBLOCKS = NNZ_PAD // 128
BLOCKS_PER_SUB = EBLOCKS // 16
EBATCH = 8
BATCHES_PER_SUB = BLOCKS_PER_SUB // EBATCH

SEQ_TOT = BATCH * SEQ
SEQ_PAD = 53248
SEQ_BLOCKS = SEQ_PAD // 128
SEQ_BLOCKS_PER_W = SEQ_BLOCKS // 32

def _sc_mesh():
    return plsc.VectorSubcoreMesh(core_axis_name="c", subcore_axis_name="s",
                                  num_cores=2, num_subcores=16)


_SC_PARAMS = pltpu.CompilerParams(use_tc_tiling_on_sc=False,
                                  needs_layout_passes=False)


def _hyperconv_sc(e0, e1, e2, e3, edges):
    f32 = jnp.float32
    bf16 = jnp.bfloat16
    i32 = jnp.int32
    out_t = jax.ShapeDtypeStruct((NCHUNK, LAYERS + 1, NROWS, W), bf16)

    NBUF = 2
    scratch_types = [pltpu.VMEM_SHARED((NROWS, W), bf16)]
    scratch_types += [pltpu.VMEM((EBATCH, 3, 128), i32)] * NBUF
    scratch_types += [pltpu.VMEM((EBATCH * 128, W), bf16)] * NBUF
    scratch_types += [pltpu.VMEM((64, W), bf16)]
    scratch_types += [pltpu.SemaphoreType.DMA] * NBUF

    @functools.partial(
        pl.kernel,
        out_type=out_t,
        mesh=_sc_mesh(),
        scratch_types=scratch_types,
        compiler_params=_SC_PARAMS,
    )
    def k(e0_h, e1_h, e2_h, e3_h, ed_h, o_h, *scr):
        acc = scr[0]
        idxb = scr[1:1 + NBUF]
        rows = scr[1 + NBUF:1 + 2 * NBUF]
        zbuf = scr[1 + 2 * NBUF]
        gsem = scr[2 + 2 * NBUF:]
        core = lax.axis_index("c")
        sid = lax.axis_index("s")

        zv = jnp.zeros((32,), bf16)

        @pl.loop(0, 64)
        def _(i):
            zbuf[i, pl.ds(0, 32)] = zv

        def zero_acc():
            base = sid * ROWS_PER_SUB

            @pl.loop(0, ROWS_PER_SUB // 64)
            def _(i):
                pltpu.sync_copy(zbuf, acc.at[pl.ds(base + i * 64, 64)])

        def edge_pass(tab):
            def fire(i, b):
                blk = sid * BLOCKS_PER_SUB + i * EBATCH
                pltpu.sync_copy(ed_h.at[pl.ds(blk, EBATCH)], idxb[b])
                for j in range(EBATCH):
                    pltpu.async_copy(tab.at[idxb[b].at[j, 0]],
                                     rows[b].at[pl.ds(j * 128, 128)], gsem[b])

            def handle(b):
                for j in range(EBATCH):
                    pltpu.make_async_copy(
                        tab.at[idxb[b].at[j, 0]],
                        rows[b].at[pl.ds(j * 128, 128)], gsem[b]).wait()

                @pl.loop(0, EBATCH * 8)
                def _(g):
                    j = g // 8
                    gg = g - j * 8
                    vv = plsc.bitcast(idxb[b][j, 2, pl.ds(gg * 16, 16)], f32)
                    e0 = g * 16
                    for t in range(16):
                        s = jnp.full((16,), vv[t], f32)
                        sb = plsc.pack(s, s,
                                       format=plsc.PackFormat.INTERLEAVED)
                        rows[b][e0 + t, pl.ds(0, 32)] = (
                            rows[b][e0 + t, pl.ds(0, 32)] * sb)

                for j in range(EBATCH):
                    pltpu.sync_copy(rows[b].at[pl.ds(j * 128, 128)],
                                    acc.at[idxb[b].at[j, 1]], add=True)

            fire(0, 0)

            @pl.loop(0, (BATCHES_PER_SUB - 1) // 2)
            def _(ii):
                i0 = ii * 2
                fire(i0 + 1, 1)
                handle(0)
                fire(i0 + 2, 0)
                handle(1)

            handle(0)

        def drain(out_h):
            base = sid * ROWS_PER_SUB
            pltpu.sync_copy(acc.at[pl.ds(base, ROWS_PER_SUB)],
                            out_h.at[pl.ds(base, ROWS_PER_SUB)])

        base = sid * ROWS_PER_SUB
        sl = pl.ds(base, ROWS_PER_SUB)

        @pl.when(core == 0)
        def _():
            pltpu.sync_copy(e0_h.at[sl], o_h.at[0, 0].at[sl])
            pltpu.sync_copy(e1_h.at[sl], o_h.at[1, 0].at[sl])

        @pl.when(core == 1)
        def _():
            pltpu.sync_copy(e2_h.at[sl], o_h.at[2, 0].at[sl])
            pltpu.sync_copy(e3_h.at[sl], o_h.at[3, 0].at[sl])

        plsc.subcore_barrier()

        @pl.loop(0, 2 * LAYERS)
        def _(q):
            cc = q // 3
            layer = q - cc * 3
            c = core * 2 + cc
            zero_acc()
            plsc.subcore_barrier()
            edge_pass(o_h.at[c, layer])
            plsc.subcore_barrier()
            drain(o_h.at[c, layer + 1])
            plsc.subcore_barrier()

    return k(e0, e1, e2, e3, edges)


def _gather_sc(tab, idx2d):
    f32 = jnp.float32

    @functools.partial(
        pl.kernel,
        out_type=jax.ShapeDtypeStruct((SEQ_PAD, EMBP), f32),
        mesh=_sc_mesh(),
        scratch_types=[
            pltpu.VMEM((128,), jnp.int32),
            pltpu.VMEM((128, EMBP), f32),
            pltpu.SemaphoreType.DMA,
        ],
        compiler_params=_SC_PARAMS,
    )
    def k(tab_h, idx_h, out_h, idxv, rowbuf, sem):
        core = lax.axis_index("c")
        sid = lax.axis_index("s")
        wid = sid * 2 + core

        @pl.loop(0, SEQ_BLOCKS_PER_W)
        def _(i):
            blk = wid * SEQ_BLOCKS_PER_W + i
            pltpu.sync_copy(idx_h.at[blk], idxv)
            pltpu.async_copy(tab_h.at[idxv], rowbuf, sem).wait()
            pltpu.sync_copy(rowbuf, out_h.at[pl.ds(blk * 128, 128)])

    return k(tab, idx2d)


def _assemble_tc(embP, O):
    RB = 2000

    def body(emb_ref, o_ref, out_ref):
        f32 = jnp.float32
        o = o_ref[...].astype(f32)
        sums = [o[c, 1] + o[c, 2] + o[c, 3] for c in range(NCHUNK)]
        cat = jnp.concatenate(
            [sums[0], sums[1], sums[2], sums[3][:, :EMBP - 3 * W]], axis=1)
        out_ref[...] = emb_ref[...] + cat

    return pl.pallas_call(
        body,
        grid=(N_NODE // RB,),
        in_specs=[
            pl.BlockSpec((RB, EMBP), lambda i: (i, 0)),
            pl.BlockSpec((NCHUNK, LAYERS + 1, RB, W),
                         lambda i: (0, 0, i, 0)),
        ],
        out_specs=pl.BlockSpec((RB, EMBP), lambda i: (i, 0)),
        out_shape=jax.ShapeDtypeStruct((N_NODE, EMBP), jnp.float32),
    )(embP, O)


def _sess_sum_tc(seqg, si_col, session_len):
    BB = 128

    def body(sq_ref, nz_ref, len_ref, out_ref):
        f32 = jnp.float32
        sq = sq_ref[...] * (nz_ref[...] != 0).astype(f32)
        out_ref[...] = jnp.sum(sq.reshape(BB, SEQ, EMBP), axis=1) / len_ref[...]

    return pl.pallas_call(
        body,
        grid=(BATCH // BB,),
        in_specs=[
            pl.BlockSpec((BB * SEQ, EMBP), lambda i: (i, 0)),
            pl.BlockSpec((BB * SEQ, 1), lambda i: (i, 0)),
            pl.BlockSpec((BB, 1), lambda i: (i, 0)),
        ],
        out_specs=pl.BlockSpec((BB, EMBP), lambda i: (i, 0)),
        out_shape=jax.ShapeDtypeStruct((BATCH, EMBP), jnp.float32),
    )(seqg, si_col, session_len)


def _lineconv_tc(D, A, s):
    def body(d_ref, a_ref, s_ref, out_ref):
        f32 = jnp.float32
        da = jnp.dot(d_ref[...], a_ref[...], preferred_element_type=f32)
        c = s_ref[...]
        acc = c
        for _ in range(LAYERS):
            c = jnp.dot(da, c, preferred_element_type=f32)
            acc = acc + c
        out_ref[...] = acc

    return pl.pallas_call(
        body,
        out_shape=jax.ShapeDtypeStruct((BATCH, EMBP), jnp.float32),
    )(D, A, s)


def _attention_tc(seqg, rsi_col, mask_col, session_len, posP,
                  w1t, w1b, glu1P, b1P, glu2P, w2P):
    BB = 128

    def body(sq_ref, nz_ref, mk_ref, len_ref, pos_ref, w1t_ref, w1b_ref,
             g1_ref, b1_ref, g2_ref, w2_ref, out_ref):
        f32 = jnp.float32
        sq = sq_ref[...] * (nz_ref[...] != 0).astype(f32)
        sq3 = sq.reshape(BB, SEQ, EMBP)
        hs = jnp.sum(sq3, axis=1) / len_ref[...]
        pos_t = jnp.dot(pos_ref[...], w1t_ref[...], preferred_element_type=f32)
        t1 = jnp.dot(sq, w1b_ref[...], preferred_element_type=f32)
        nh = jnp.tanh(t1.reshape(BB, SEQ, EMBP) + pos_t[None])
        hsg = jnp.dot(hs, g2_ref[...], preferred_element_type=f32)
        g1 = jnp.dot(nh.reshape(BB * SEQ, EMBP), g1_ref[...],
                     preferred_element_type=f32)
        g = jax.nn.sigmoid(g1.reshape(BB, SEQ, EMBP) + b1_ref[...][None]
                           + hsg[:, None, :])
        beta = jnp.sum(g * w2_ref[...][None], axis=-1, keepdims=True)
        beta = beta * mk_ref[...].reshape(BB, SEQ, 1)
        out_ref[...] = jnp.sum(beta * sq3, axis=1)

    return pl.pallas_call(
        body,
        grid=(BATCH // BB,),
        in_specs=[
            pl.BlockSpec((BB * SEQ, EMBP), lambda i: (i, 0)),
            pl.BlockSpec((BB * SEQ, 1), lambda i: (i, 0)),
            pl.BlockSpec((BB * SEQ, 1), lambda i: (i, 0)),
            pl.BlockSpec((BB, 1), lambda i: (i, 0)),
            pl.BlockSpec((SEQ, EMBP), lambda i: (0, 0)),
            pl.BlockSpec((EMBP, EMBP), lambda i: (0, 0)),
            pl.BlockSpec((EMBP, EMBP), lambda i: (0, 0)),
            pl.BlockSpec((EMBP, EMBP), lambda i: (0, 0)),
            pl.BlockSpec((1, EMBP), lambda i: (0, 0)),
            pl.BlockSpec((EMBP, EMBP), lambda i: (0, 0)),
            pl.BlockSpec((1, EMBP), lambda i: (0, 0)),
        ],
        out_specs=pl.BlockSpec((BB, EMBP), lambda i: (i, 0)),
        out_shape=jax.ShapeDtypeStruct((BATCH, EMBP), jnp.float32),
    )(seqg, rsi_col, mask_col, session_len, posP, w1t, w1b, glu1P, b1P,
      glu2P, w2P)


def _loss_tc(se, sl, Pr, Pc):
    def body(se_ref, sl_ref, pr_ref, pc_ref, out_ref):
        f32 = jnp.float32
        se_v = se_ref[...]
        sl_v = sl_ref[...]
        corrupt = jnp.dot(
            jnp.dot(pr_ref[...], se_v, preferred_element_type=f32),
            pc_ref[...], preferred_element_type=f32)
        pos = jnp.sum(se_v * sl_v, axis=1, keepdims=True)
        neg = jnp.sum(sl_v * corrupt, axis=1, keepdims=True)
        term = (-jnp.log(1e-08 + jax.nn.sigmoid(pos))
                - jnp.log(1e-08 + (1.0 - jax.nn.sigmoid(neg))))
        out_ref[...] = (BETA * jnp.sum(term)).reshape(1, 1)

    return pl.pallas_call(
        body,
        out_shape=jax.ShapeDtypeStruct((1, 1), jnp.float32),
    )(se, sl, Pr, Pc)


def kernel(session_item, session_len, D, A, reversed_sess_item, mask,
           adj_row, adj_col, adj_val, embedding, pos_embedding,
           w_1, w_2, glu1_w, glu1_b, glu2_w):
    f32 = jnp.float32
    i32 = jnp.int32

    emb128 = jnp.pad(embedding, ((0, 0), (0, 4 * W - EMB)))
    emb128b = jnp.pad(emb128.astype(jnp.bfloat16),
                      ((0, NROWS - N_NODE), (0, 0)))
    e_chunks = [emb128b[:, c * W:(c + 1) * W] for c in range(NCHUNK)]
    embP = emb128[:, :EMBP]

    padE = NNZ_PAD - NNZ
    rowp = jnp.concatenate(
        [adj_row, jnp.full((padE,), N_NODE, i32)]).reshape(EBLOCKS, 128)
    colp = jnp.concatenate(
        [adj_col, jnp.zeros((padE,), i32)]).reshape(EBLOCKS, 128)
    valp = jax.lax.bitcast_convert_type(
        jnp.concatenate([adj_val, jnp.zeros((padE,), f32)]),
        i32).reshape(EBLOCKS, 128)
    edges = jnp.stack([colp, rowp, valp], axis=1)

    idx_pad = jnp.zeros((SEQ_PAD - SEQ_TOT,), i32)
    si_flat = session_item.reshape(-1)
    si_idx = jnp.concatenate(
        [jnp.maximum(si_flat - 1, 0), idx_pad]).reshape(SEQ_BLOCKS, 128)
    rsi_flat = reversed_sess_item.reshape(-1)
    rsi_idx = jnp.concatenate(
        [jnp.maximum(rsi_flat - 1, 0), idx_pad]).reshape(SEQ_BLOCKS, 128)

    pad12 = EMBP - EMB
    w1t = jnp.pad(w_1[:EMB], ((0, pad12), (0, pad12)))
    w1b = jnp.pad(w_1[EMB:], ((0, pad12), (0, pad12)))
    glu1P = jnp.pad(glu1_w, ((0, pad12), (0, pad12)))
    glu2P = jnp.pad(glu2_w, ((0, pad12), (0, pad12)))
    b1P = jnp.pad(glu1_b, (0, pad12)).reshape(1, EMBP)
    w2P = jnp.pad(w_2[:, 0], (0, pad12)).reshape(1, EMBP)
    posP = jnp.pad(pos_embedding[:SEQ], ((0, 0), (0, pad12)))

    pr = jax.random.permutation(jax.random.key(123), BATCH)
    pc = jax.random.permutation(jax.random.key(456), EMB)
    Pr = jax.nn.one_hot(pr, BATCH, dtype=f32)
    Pc = jnp.pad(jax.nn.one_hot(pc, EMBP, dtype=f32).T, ((0, 0), (0, pad12)))

    seq2g = _gather_sc(embP, si_idx)
    hc_out = _hyperconv_sc(*e_chunks, edges)

    s = _sess_sum_tc(seq2g, si_flat.reshape(-1, 1), session_len)
    sess_lg = _lineconv_tc(D, A, s)

    item_hgP = _assemble_tc(embP, hc_out)
    seqhg = _gather_sc(item_hgP, rsi_idx)
    sess_emb = _attention_tc(seqhg, rsi_flat.reshape(-1, 1),
                             mask.reshape(-1, 1), session_len, posP,
                             w1t, w1b, glu1P, b1P, glu2P, w2P)
    loss = _loss_tc(sess_emb, sess_lg, Pr, Pc)

    return item_hgP[:, :EMB], sess_emb[:, :EMB], loss.reshape(())


# speedup vs baseline: 1.1336x; 1.0940x over previous
import functools

import jax
import jax.numpy as jnp
from jax import lax
from jax.experimental import pallas as pl
from jax.experimental.pallas import tpu as pltpu
from jax.experimental.pallas import tpu_sc as plsc

N_NODE = 50000
EMB = 100
EMBP = 112
LAYERS = 3
BATCH = 1024
SEQ = 50
NNZ = 800000
BETA = 0.01

W = 32
NCHUNK = 4
NROWS = 50176
ROWS_PER_SUB = NROWS // 16
EDGES_PER_SUB = 50176
NNZ_PAD = 16 * EDGES_PER_SUB
EBLOCKS = NNZ_PAD // 128
BLOCKS_PER_SUB = EBLOCKS // 16
EBATCH = 8
BATCHES_PER_SUB = BLOCKS_PER_SUB // EBATCH

SEQ_TOT = BATCH * SEQ
SEQ_PAD = 53248
SEQ_BLOCKS = SEQ_PAD // 128
SEQ_BLOCKS_PER_W = SEQ_BLOCKS // 32

def _sc_mesh():
    return plsc.VectorSubcoreMesh(core_axis_name="c", subcore_axis_name="s",
                                  num_cores=2, num_subcores=16)


_SC_PARAMS = pltpu.CompilerParams(use_tc_tiling_on_sc=False,
                                  needs_layout_passes=False)


def _hyperconv_sc(e0, e1, e2, e3, row2d, col2d, val1d):
    f32 = jnp.float32
    bf16 = jnp.bfloat16
    i32 = jnp.int32
    out_t = [jax.ShapeDtypeStruct((NROWS, W), bf16) for _ in range(12)]

    @functools.partial(
        pl.kernel,
        out_type=out_t,
        mesh=_sc_mesh(),
        scratch_types=[
            pltpu.VMEM_SHARED((NROWS, W), bf16),
            pltpu.VMEM((EBATCH, 128), i32),
            pltpu.VMEM((EBATCH, 128), i32),
            pltpu.VMEM((EBATCH, 128), i32),
            pltpu.VMEM((EBATCH, 128), i32),
            pltpu.VMEM((EBATCH * 128,), f32),
            pltpu.VMEM((EBATCH * 128,), f32),
            pltpu.VMEM((EBATCH * 128, W), bf16),
            pltpu.VMEM((EBATCH * 128, W), bf16),
            pltpu.VMEM((128, W), bf16),
            pltpu.SemaphoreType.DMA,
            pltpu.SemaphoreType.DMA,
        ],
        compiler_params=_SC_PARAMS,
    )
    def k(e0_h, e1_h, e2_h, e3_h, row_h, col_h, val_h, *rest):
        outs = rest[:12]
        (acc, colv0, colv1, rowv0, rowv1, valv0, valv1, rows0, rows1,
         zbuf, sem0, sem1) = rest[12:]
        colv = (colv0, colv1)
        rowv = (rowv0, rowv1)
        valv = (valv0, valv1)
        rows = (rows0, rows1)
        gsem = (sem0, sem1)
        core = lax.axis_index("c")
        sid = lax.axis_index("s")

        zv = jnp.zeros((32,), bf16)

        @pl.loop(0, 128)
        def _(i):
            zbuf[i, pl.ds(0, 32)] = zv

        def zero_acc():
            base = sid * ROWS_PER_SUB

            @pl.loop(0, ROWS_PER_SUB // 128)
            def _(i):
                pltpu.sync_copy(zbuf, acc.at[pl.ds(base + i * 128, 128)])

            rem = ROWS_PER_SUB % 128
            if rem:
                pltpu.sync_copy(
                    zbuf.at[pl.ds(0, rem)],
                    acc.at[pl.ds(base + (ROWS_PER_SUB // 128) * 128, rem)])

        def edge_pass(tab):
            def fire(i, b):
                blk = sid * BLOCKS_PER_SUB + i * EBATCH
                pltpu.sync_copy(col_h.at[pl.ds(blk, EBATCH)], colv[b])
                pltpu.sync_copy(row_h.at[pl.ds(blk, EBATCH)], rowv[b])
                pltpu.sync_copy(val_h.at[pl.ds(blk * 128, EBATCH * 128)],
                                valv[b])
                for j in range(EBATCH):
                    pltpu.async_copy(tab.at[colv[b].at[j]],
                                     rows[b].at[pl.ds(j * 128, 128)], gsem[b])

            def handle(b):
                for j in range(EBATCH):
                    pltpu.make_async_copy(
                        tab.at[colv[b].at[j]],
                        rows[b].at[pl.ds(j * 128, 128)], gsem[b]).wait()

                @pl.loop(0, EBATCH * 8)
                def _(g):
                    vv = valv[b][pl.ds(g * 16, 16)]
                    e0 = g * 16
                    for t in range(16):
                        s = jnp.full((16,), vv[t], f32)
                        sb = plsc.pack(s, s,
                                       format=plsc.PackFormat.INTERLEAVED)
                        rows[b][e0 + t, pl.ds(0, 32)] = (
                            rows[b][e0 + t, pl.ds(0, 32)] * sb)

                for j in range(EBATCH):
                    pltpu.sync_copy(rows[b].at[pl.ds(j * 128, 128)],
                                    acc.at[rowv[b].at[j]], add=True)

            fire(0, 0)

            @pl.loop(0, (BATCHES_PER_SUB - 1) // 2)
            def _(ii):
                i0 = ii * 2
                fire(i0 + 1, 1)
                handle(0)
                fire(i0 + 2, 0)
                handle(1)

            handle(0)

        def drain(out_h):
            base = sid * ROWS_PER_SUB
            pltpu.sync_copy(acc.at[pl.ds(base, ROWS_PER_SUB)],
                            out_h.at[pl.ds(base, ROWS_PER_SUB)])

        def run_core(tables2, outs2):
            for cc in range(2):
                tabs = [tables2[cc], outs2[cc][0], outs2[cc][1]]
                for layer in range(LAYERS):
                    zero_acc()
                    plsc.subcore_barrier()
                    edge_pass(tabs[layer])
                    plsc.subcore_barrier()
                    drain(outs2[cc][layer])
                    plsc.subcore_barrier()

        o = [[outs[c * 3 + l] for l in range(LAYERS)] for c in range(NCHUNK)]

        @pl.when(core == 0)
        def _():
            run_core([e0_h, e1_h], [o[0], o[1]])

        @pl.when(core == 1)
        def _():
            run_core([e2_h, e3_h], [o[2], o[3]])

    return k(e0, e1, e2, e3, row2d, col2d, val1d)


def _gather_sc(tab, idx2d):
    f32 = jnp.float32

    @functools.partial(
        pl.kernel,
        out_type=jax.ShapeDtypeStruct((SEQ_PAD, EMBP), f32),
        mesh=_sc_mesh(),
        scratch_types=[
            pltpu.VMEM((128,), jnp.int32),
            pltpu.VMEM((128, EMBP), f32),
            pltpu.SemaphoreType.DMA,
        ],
        compiler_params=_SC_PARAMS,
    )
    def k(tab_h, idx_h, out_h, idxv, rowbuf, sem):
        core = lax.axis_index("c")
        sid = lax.axis_index("s")
        wid = sid * 2 + core

        @pl.loop(0, SEQ_BLOCKS_PER_W)
        def _(i):
            blk = wid * SEQ_BLOCKS_PER_W + i
            pltpu.sync_copy(idx_h.at[blk], idxv)
            pltpu.async_copy(tab_h.at[idxv], rowbuf, sem).wait()
            pltpu.sync_copy(rowbuf, out_h.at[pl.ds(blk * 128, 128)])

    return k(tab, idx2d)


def _assemble_tc(embP, curs):
    RB = 2000

    def body(emb_ref, *refs):
        cur_refs, out_ref = refs[:12], refs[12]
        f32 = jnp.float32
        sums = [
            cur_refs[c * 3][...].astype(f32)
            + cur_refs[c * 3 + 1][...].astype(f32)
            + cur_refs[c * 3 + 2][...].astype(f32)
            for c in range(NCHUNK)
        ]
        cat = jnp.concatenate(
            [sums[0], sums[1], sums[2], sums[3][:, :EMBP - 3 * W]], axis=1)
        out_ref[...] = emb_ref[...] + cat

    return pl.pallas_call(
        body,
        grid=(N_NODE // RB,),
        in_specs=[pl.BlockSpec((RB, EMBP), lambda i: (i, 0))] +
                 [pl.BlockSpec((RB, W), lambda i: (i, 0))] * 12,
        out_specs=pl.BlockSpec((RB, EMBP), lambda i: (i, 0)),
        out_shape=jax.ShapeDtypeStruct((N_NODE, EMBP), jnp.float32),
    )(embP, *curs)


def _sess_sum_tc(seqg, si_col, session_len):
    BB = 128

    def body(sq_ref, nz_ref, len_ref, out_ref):
        f32 = jnp.float32
        sq = sq_ref[...] * (nz_ref[...] != 0).astype(f32)
        out_ref[...] = jnp.sum(sq.reshape(BB, SEQ, EMBP), axis=1) / len_ref[...]

    return pl.pallas_call(
        body,
        grid=(BATCH // BB,),
        in_specs=[
            pl.BlockSpec((BB * SEQ, EMBP), lambda i: (i, 0)),
            pl.BlockSpec((BB * SEQ, 1), lambda i: (i, 0)),
            pl.BlockSpec((BB, 1), lambda i: (i, 0)),
        ],
        out_specs=pl.BlockSpec((BB, EMBP), lambda i: (i, 0)),
        out_shape=jax.ShapeDtypeStruct((BATCH, EMBP), jnp.float32),
    )(seqg, si_col, session_len)


def _lineconv_tc(D, A, s):
    def body(d_ref, a_ref, s_ref, out_ref):
        f32 = jnp.float32
        da = jnp.dot(d_ref[...], a_ref[...], preferred_element_type=f32)
        c = s_ref[...]
        acc = c
        for _ in range(LAYERS):
            c = jnp.dot(da, c, preferred_element_type=f32)
            acc = acc + c
        out_ref[...] = acc

    return pl.pallas_call(
        body,
        out_shape=jax.ShapeDtypeStruct((BATCH, EMBP), jnp.float32),
    )(D, A, s)


def _attention_tc(seqg, rsi_col, mask_col, session_len, posP,
                  w1t, w1b, glu1P, b1P, glu2P, w2P):
    BB = 128

    def body(sq_ref, nz_ref, mk_ref, len_ref, pos_ref, w1t_ref, w1b_ref,
             g1_ref, b1_ref, g2_ref, w2_ref, out_ref):
        f32 = jnp.float32
        sq = sq_ref[...] * (nz_ref[...] != 0).astype(f32)
        sq3 = sq.reshape(BB, SEQ, EMBP)
        hs = jnp.sum(sq3, axis=1) / len_ref[...]
        pos_t = jnp.dot(pos_ref[...], w1t_ref[...], preferred_element_type=f32)
        t1 = jnp.dot(sq, w1b_ref[...], preferred_element_type=f32)
        nh = jnp.tanh(t1.reshape(BB, SEQ, EMBP) + pos_t[None])
        hsg = jnp.dot(hs, g2_ref[...], preferred_element_type=f32)
        g1 = jnp.dot(nh.reshape(BB * SEQ, EMBP), g1_ref[...],
                     preferred_element_type=f32)
        g = jax.nn.sigmoid(g1.reshape(BB, SEQ, EMBP) + b1_ref[...][None]
                           + hsg[:, None, :])
        beta = jnp.sum(g * w2_ref[...][None], axis=-1, keepdims=True)
        beta = beta * mk_ref[...].reshape(BB, SEQ, 1)
        out_ref[...] = jnp.sum(beta * sq3, axis=1)

    return pl.pallas_call(
        body,
        grid=(BATCH // BB,),
        in_specs=[
            pl.BlockSpec((BB * SEQ, EMBP), lambda i: (i, 0)),
            pl.BlockSpec((BB * SEQ, 1), lambda i: (i, 0)),
            pl.BlockSpec((BB * SEQ, 1), lambda i: (i, 0)),
            pl.BlockSpec((BB, 1), lambda i: (i, 0)),
            pl.BlockSpec((SEQ, EMBP), lambda i: (0, 0)),
            pl.BlockSpec((EMBP, EMBP), lambda i: (0, 0)),
            pl.BlockSpec((EMBP, EMBP), lambda i: (0, 0)),
            pl.BlockSpec((EMBP, EMBP), lambda i: (0, 0)),
            pl.BlockSpec((1, EMBP), lambda i: (0, 0)),
            pl.BlockSpec((EMBP, EMBP), lambda i: (0, 0)),
            pl.BlockSpec((1, EMBP), lambda i: (0, 0)),
        ],
        out_specs=pl.BlockSpec((BB, EMBP), lambda i: (i, 0)),
        out_shape=jax.ShapeDtypeStruct((BATCH, EMBP), jnp.float32),
    )(seqg, rsi_col, mask_col, session_len, posP, w1t, w1b, glu1P, b1P,
      glu2P, w2P)


def _loss_tc(se, sl, Pr, Pc):
    def body(se_ref, sl_ref, pr_ref, pc_ref, out_ref):
        f32 = jnp.float32
        se_v = se_ref[...]
        sl_v = sl_ref[...]
        corrupt = jnp.dot(
            jnp.dot(pr_ref[...], se_v, preferred_element_type=f32),
            pc_ref[...], preferred_element_type=f32)
        pos = jnp.sum(se_v * sl_v, axis=1, keepdims=True)
        neg = jnp.sum(sl_v * corrupt, axis=1, keepdims=True)
        term = (-jnp.log(1e-08 + jax.nn.sigmoid(pos))
                - jnp.log(1e-08 + (1.0 - jax.nn.sigmoid(neg))))
        out_ref[...] = (BETA * jnp.sum(term)).reshape(1, 1)

    return pl.pallas_call(
        body,
        out_shape=jax.ShapeDtypeStruct((1, 1), jnp.float32),
    )(se, sl, Pr, Pc)


def kernel(session_item, session_len, D, A, reversed_sess_item, mask,
           adj_row, adj_col, adj_val, embedding, pos_embedding,
           w_1, w_2, glu1_w, glu1_b, glu2_w):
    f32 = jnp.float32
    i32 = jnp.int32

    emb128 = jnp.pad(embedding, ((0, 0), (0, 4 * W - EMB)))
    emb128b = jnp.pad(emb128.astype(jnp.bfloat16),
                      ((0, NROWS - N_NODE), (0, 0)))
    e_chunks = [emb128b[:, c * W:(c + 1) * W] for c in range(NCHUNK)]
    embP = emb128[:, :EMBP]

    padE = NNZ_PAD - NNZ
    rowp = jnp.concatenate(
        [adj_row, jnp.full((padE,), N_NODE, i32)]).reshape(EBLOCKS, 128)
    colp = jnp.concatenate(
        [adj_col, jnp.zeros((padE,), i32)]).reshape(EBLOCKS, 128)
    valp = jnp.concatenate([adj_val, jnp.zeros((padE,), f32)])

    idx_pad = jnp.zeros((SEQ_PAD - SEQ_TOT,), i32)
    si_flat = session_item.reshape(-1)
    si_idx = jnp.concatenate(
        [jnp.maximum(si_flat - 1, 0), idx_pad]).reshape(SEQ_BLOCKS, 128)
    rsi_flat = reversed_sess_item.reshape(-1)
    rsi_idx = jnp.concatenate(
        [jnp.maximum(rsi_flat - 1, 0), idx_pad]).reshape(SEQ_BLOCKS, 128)

    pad12 = EMBP - EMB
    w1t = jnp.pad(w_1[:EMB], ((0, pad12), (0, pad12)))
    w1b = jnp.pad(w_1[EMB:], ((0, pad12), (0, pad12)))
    glu1P = jnp.pad(glu1_w, ((0, pad12), (0, pad12)))
    glu2P = jnp.pad(glu2_w, ((0, pad12), (0, pad12)))
    b1P = jnp.pad(glu1_b, (0, pad12)).reshape(1, EMBP)
    w2P = jnp.pad(w_2[:, 0], (0, pad12)).reshape(1, EMBP)
    posP = jnp.pad(pos_embedding[:SEQ], ((0, 0), (0, pad12)))

    pr = jax.random.permutation(jax.random.key(123), BATCH)
    pc = jax.random.permutation(jax.random.key(456), EMB)
    Pr = jax.nn.one_hot(pr, BATCH, dtype=f32)
    Pc = jnp.pad(jax.nn.one_hot(pc, EMBP, dtype=f32).T, ((0, 0), (0, pad12)))

    seq2g = _gather_sc(embP, si_idx)
    hc_out = _hyperconv_sc(*e_chunks, rowp, colp, valp)

    s = _sess_sum_tc(seq2g, si_flat.reshape(-1, 1), session_len)
    sess_lg = _lineconv_tc(D, A, s)

    item_hgP = _assemble_tc(embP, hc_out)
    seqhg = _gather_sc(item_hgP, rsi_idx)
    sess_emb = _attention_tc(seqhg, rsi_flat.reshape(-1, 1),
                             mask.reshape(-1, 1), session_len, posP,
                             w1t, w1b, glu1P, b1P, glu2P, w2P)
    loss = _loss_tc(sess_emb, sess_lg, Pr, Pc)

    return item_hgP[:, :EMB], sess_emb[:, :EMB], loss.reshape(())
